# baseline (speedup 1.0000x reference)
import jax
import jax.numpy as jnp
from jax.experimental import pallas as pl
from jax.experimental.pallas import tpu as pltpu

B, N, D_FEAT, D_INNER, D_ATT = 4, 4096, 1024, 1024, 128
N1, N2, N3, N_CLASS = 1, 2, 2, 2
TS = 512
S = 4
TN = TS * S
NT = N // TN
HIGHEST = jax.lax.Precision.HIGHEST
BF = jnp.bfloat16


def _dot_t(a, b):
    return jax.lax.dot_general(a, b, (((1,), (1,)), ((), ())),
                               preferred_element_type=jnp.float32)


def _split_rows(ref, w, r0, d):
    hi = w.astype(BF)
    lo = (w - hi.astype(jnp.float32)).astype(BF)
    rows = w.shape[0]
    ref[pl.ds(r0, rows), 0:d] = hi
    ref[pl.ds(r0, rows), d:2 * d] = lo
    ref[pl.ds(r0, rows), 2 * d:3 * d] = hi


def _cat_lhs(ref, a, d):
    hi = a.astype(BF)
    lo = (a - hi.astype(jnp.float32)).astype(BF)
    ref[:, 0:d] = hi
    ref[:, d:2 * d] = hi
    ref[:, 2 * d:3 * d] = lo


def _stage1_body(x_ref, w1_ref, vw_ref, uw_ref, vub_ref, aw_ref, ab_ref,
                 logits_ref, afeat_ref, m_ref, s_ref, acc_ref,
                 fc0_ref, fc1_ref, xc0_ref, xc1_ref, w1c_ref, vuc_ref):
    b = pl.program_id(0)
    t = pl.program_id(1)

    @pl.when(jnp.logical_and(b == 0, t == 0))
    def _():
        _split_rows(w1c_ref, w1_ref[...], 0, D_FEAT)
        _split_rows(vuc_ref, vw_ref[...], 0, D_INNER)
        _split_rows(vuc_ref, uw_ref[...], D_ATT, D_INNER)

    @pl.when(t == 0)
    def _():
        m_ref[...] = jnp.full_like(m_ref, -jnp.inf)
        s_ref[...] = jnp.zeros_like(s_ref)
        acc_ref[...] = jnp.zeros_like(acc_ref)

    for i in range(S):
        fc = fc0_ref if i % 2 == 0 else fc1_ref
        xc = xc0_ref if i % 2 == 0 else xc1_ref
        feat = x_ref[0, 0, pl.ds(i * TS, TS), :]
        _cat_lhs(fc, feat, D_FEAT)
        xr = jax.nn.relu(_dot_t(fc[...], w1c_ref[...]))
        _cat_lhs(xc, xr, D_INNER)
        avu = _dot_t(xc[...], vuc_ref[...]) + vub_ref[...]
        h = jnp.tanh(avu[:, :D_ATT]) * jax.nn.sigmoid(avu[:, D_ATT:])
        l = jax.lax.dot_general(aw_ref[...], h, (((1,), (1,)), ((), ())),
                                preferred_element_type=jnp.float32,
                                precision=HIGHEST) + ab_ref[...]
        logits_ref[0, 0:1, pl.ds(t * TN + i * TS, TS)] = l

        m_t = jnp.max(l, axis=1, keepdims=True)
        m_old = m_ref[...]
        m_new = jnp.maximum(m_old, m_t)
        corr = jnp.exp(m_old - m_new)
        p = jnp.exp(l - m_new)
        s_ref[...] = s_ref[...] * corr + jnp.sum(p, axis=1, keepdims=True)
        p_bf = p.astype(BF)
        pooled = (jax.lax.dot_general(p_bf, fc[:, 0:D_FEAT],
                                      (((1,), (0,)), ((), ())),
                                      preferred_element_type=jnp.float32)
                  + jax.lax.dot_general(p_bf, fc[:, 2 * D_FEAT:],
                                        (((1,), (0,)), ((), ())),
                                        preferred_element_type=jnp.float32))
        acc_ref[...] = acc_ref[...] * corr + pooled
        m_ref[...] = m_new

    @pl.when(t == NT - 1)
    def _():
        afeat_ref[0] = acc_ref[...] / s_ref[...]


def _stage2_body(logits_ref, afeat_ref, w2_ref, a2vw_ref, a2uw_ref, a2vub_ref,
                 a2w_ref, a2b_ref, a3vw_ref, a3uw_ref, a3vub_ref,
                 a3w_ref, a3b_ref,
                 clsw0_ref, clsw1_ref, clsb_ref, slw_ref, slb_ref,
                 a1_ref, a2_ref, out_ref, slide_ref,
                 ac_ref, yc_ref, bc_ref, w2c_ref, vu2c_ref, vu3c_ref):
    _split_rows(w2c_ref, w2_ref[...], 0, D_FEAT)
    _split_rows(vu2c_ref, a2vw_ref[...], 0, D_INNER)
    _split_rows(vu2c_ref, a2uw_ref[...], D_ATT, D_INNER)
    _split_rows(vu3c_ref, a3vw_ref[...], 0, D_FEAT)
    _split_rows(vu3c_ref, a3uw_ref[...], D_ATT, D_FEAT)

    l = logits_ref[...]
    m = jnp.max(l, axis=1, keepdims=True)
    p = jnp.exp(l - m)
    a1_ref[...] = p / jnp.sum(p, axis=1, keepdims=True)

    afeat = afeat_ref[...]
    _cat_lhs(ac_ref, afeat, D_FEAT)
    y = jax.nn.relu(_dot_t(ac_ref[...], w2c_ref[...]))
    _cat_lhs(yc_ref, y, D_INNER)
    avu = _dot_t(yc_ref[...], vu2c_ref[...]) + a2vub_ref[...]
    h = jnp.tanh(avu[:, :D_ATT]) * jax.nn.sigmoid(avu[:, D_ATT:])
    a2p = jax.lax.dot_general(a2w_ref[...], h, (((1,), (1,)), ((), ())),
                              preferred_element_type=jnp.float32,
                              precision=HIGHEST) + a2b_ref[...]
    m2 = jnp.max(a2p, axis=1, keepdims=True)
    e2 = jnp.exp(a2p - m2)
    a2 = e2 / jnp.sum(e2, axis=1, keepdims=True)
    a2_ref[...] = a2

    afeat2 = jax.lax.dot_general(a2, afeat, (((1,), (0,)), ((), ())),
                                 preferred_element_type=jnp.float32,
                                 precision=HIGHEST)
    o0 = jnp.sum(afeat2 * clsw0_ref[...], axis=1, keepdims=True)
    o1 = jnp.sum(afeat2 * clsw1_ref[...], axis=1, keepdims=True)
    out_ref[...] = jnp.concatenate([o0, o1], axis=1) + clsb_ref[...]

    ii = jax.lax.broadcasted_iota(jnp.int32, (2, N2 + N3), 0)
    jj = jax.lax.broadcasted_iota(jnp.int32, (2, N2 + N3), 1)
    mix = jnp.where(((ii == 0) & (jj >= N2)) | ((ii == 1) & (jj < N2)),
                    0.5, 0.0)
    bag_a = jax.lax.dot_general(mix, a2, (((1,), (0,)), ((), ())),
                                preferred_element_type=jnp.float32,
                                precision=HIGHEST)
    bag_feat = jax.lax.dot_general(bag_a, afeat, (((1,), (0,)), ((), ())),
                                   preferred_element_type=jnp.float32,
                                   precision=HIGHEST)
    _cat_lhs(bc_ref, bag_feat, D_FEAT)
    avu3 = _dot_t(bc_ref[...], vu3c_ref[...]) + a3vub_ref[...]
    h3 = jnp.tanh(avu3[:, :D_ATT]) * jax.nn.sigmoid(avu3[:, D_ATT:])
    a3p = jax.lax.dot_general(a3w_ref[...], h3, (((1,), (1,)), ((), ())),
                              preferred_element_type=jnp.float32,
                              precision=HIGHEST) + a3b_ref[...]
    m3 = jnp.max(a3p, axis=1, keepdims=True)
    e3 = jnp.exp(a3p - m3)
    a3 = e3 / jnp.sum(e3, axis=1, keepdims=True)
    fb = jax.lax.dot_general(a3, bag_feat, (((1,), (0,)), ((), ())),
                             preferred_element_type=jnp.float32,
                             precision=HIGHEST)
    slide_ref[...] = jax.lax.dot_general(
        fb, slw_ref[...], (((1,), (1,)), ((), ())),
        preferred_element_type=jnp.float32, precision=HIGHEST) + slb_ref[...]


def kernel(x, W_dr1, W_dr2, a1_Vw, a1_Vb, a1_Uw, a1_Ub, a1_w, a1_b,
           a2_Vw, a2_Vb, a2_Uw, a2_Ub, a2_w, a2_b,
           a3_Vw, a3_Vb, a3_Uw, a3_Ub, a3_w, a3_b,
           cls_w, cls_b, slide_w, slide_b):
    f32 = jnp.float32
    whole = lambda shape: pl.BlockSpec(shape, lambda b, t: (0,) * len(shape))

    stage1 = pl.pallas_call(
        _stage1_body,
        grid=(B, NT),
        in_specs=[
            pl.BlockSpec((1, 1, TN, D_FEAT), lambda b, t: (0, b, t, 0)),
            whole((D_INNER, D_FEAT)),
            whole((D_ATT, D_INNER)),
            whole((D_ATT, D_INNER)),
            whole((1, 2 * D_ATT)),
            whole((N1, D_ATT)),
            whole((1, N1)),
        ],
        out_specs=[
            pl.BlockSpec((1, 1, N), lambda b, t: (b, 0, 0)),
            pl.BlockSpec((1, 1, D_FEAT), lambda b, t: (b, 0, 0)),
        ],
        out_shape=[
            jax.ShapeDtypeStruct((B, 1, N), f32),
            jax.ShapeDtypeStruct((B, 1, D_FEAT), f32),
        ],
        scratch_shapes=[
            pltpu.VMEM((1, 1), f32),
            pltpu.VMEM((1, 1), f32),
            pltpu.VMEM((1, D_FEAT), f32),
            pltpu.VMEM((TS, 3 * D_FEAT), BF),
            pltpu.VMEM((TS, 3 * D_FEAT), BF),
            pltpu.VMEM((TS, 3 * D_INNER), BF),
            pltpu.VMEM((TS, 3 * D_INNER), BF),
            pltpu.VMEM((D_INNER, 3 * D_FEAT), BF),
            pltpu.VMEM((2 * D_ATT, 3 * D_INNER), BF),
        ],
        compiler_params=pltpu.CompilerParams(
            dimension_semantics=("arbitrary", "arbitrary")),
    )
    vub = jnp.concatenate([a1_Vb, a1_Ub], axis=0).reshape(1, 2 * D_ATT)
    logits, afeat = stage1(x, W_dr1, a1_Vw, a1_Uw, vub,
                           a1_w, a1_b.reshape(1, N1))

    logits = logits.reshape(B, N)
    afeat = afeat.reshape(B, D_FEAT)

    T = N2 + N3
    vub2 = jnp.concatenate([a2_Vb, a2_Ub], axis=0).reshape(1, 2 * D_ATT)
    vub3 = jnp.concatenate([a3_Vb, a3_Ub], axis=0).reshape(1, 2 * D_ATT)
    a1_out, a2_out, outputs, slide = pl.pallas_call(
        _stage2_body,
        out_shape=[
            jax.ShapeDtypeStruct((B, N), f32),
            jax.ShapeDtypeStruct((T, B), f32),
            jax.ShapeDtypeStruct((T, N_CLASS), f32),
            jax.ShapeDtypeStruct((1, N_CLASS), f32),
        ],
        scratch_shapes=[
            pltpu.VMEM((B, 3 * D_FEAT), BF),
            pltpu.VMEM((B, 3 * D_INNER), BF),
            pltpu.VMEM((2, 3 * D_FEAT), BF),
            pltpu.VMEM((D_INNER, 3 * D_FEAT), BF),
            pltpu.VMEM((2 * D_ATT, 3 * D_INNER), BF),
            pltpu.VMEM((2 * D_ATT, 3 * D_FEAT), BF),
        ],
    )(logits, afeat, W_dr2, a2_Vw, a2_Uw, vub2, a2_w, a2_b.reshape(T, 1),
      a3_Vw, a3_Uw, vub3, a3_w, a3_b.reshape(1, 1),
      cls_w[:, 0, :], cls_w[:, 1, :], cls_b, slide_w, slide_b.reshape(1, N_CLASS))

    A_1 = a1_out.reshape(B, N1, N)
    return (outputs, slide, A_1, a2_out[:N2], a2_out[N2:], a2_out)


# speedup vs baseline: 1.4750x; 1.4750x over previous
import jax
import jax.numpy as jnp
from jax.experimental import pallas as pl
from jax.experimental.pallas import tpu as pltpu

B, N, D_FEAT, D_INNER, D_ATT = 4, 4096, 1024, 1024, 128
N1, N2, N3, N_CLASS = 1, 2, 2, 2
TS = 512
S = 4
TN = TS * S
NT = N // TN
HIGHEST = jax.lax.Precision.HIGHEST
BF = jnp.bfloat16
F32 = jnp.float32


def _dot_t(a, b):
    return jax.lax.dot_general(a, b, (((1,), (1,)), ((), ())),
                               preferred_element_type=F32)


def _wsplit(ref_h, ref_l, w, r0):
    hi = w.astype(BF)
    rows = w.shape[0]
    ref_h[pl.ds(r0, rows), :] = hi.astype(F32)
    ref_l[pl.ds(r0, rows), :] = (w - hi.astype(F32)).astype(BF)


def _wdot(a, ref_h, ref_l):
    return (_dot_t(a, ref_h[...]) + _dot_t(a.astype(BF), ref_l[...]))


def _stage1_body(x_ref, w1_ref, vw_ref, uw_ref, vub_ref, aw_ref, ab_ref,
                 logits_ref, afeat_ref, m_ref, s_ref, acc_ref,
                 w1h_ref, w1l_ref, vuh_ref, vul_ref):
    b = pl.program_id(0)
    t = pl.program_id(1)

    @pl.when(jnp.logical_and(b == 0, t == 0))
    def _():
        _wsplit(w1h_ref, w1l_ref, w1_ref[...], 0)
        _wsplit(vuh_ref, vul_ref, vw_ref[...], 0)
        _wsplit(vuh_ref, vul_ref, uw_ref[...], D_ATT)

    @pl.when(t == 0)
    def _():
        m_ref[...] = jnp.full_like(m_ref, -jnp.inf)
        s_ref[...] = jnp.zeros_like(s_ref)
        acc_ref[...] = jnp.zeros_like(acc_ref)

    for i in range(S):
        feat = x_ref[0, 0, pl.ds(i * TS, TS), :]
        xr = jax.nn.relu(_wdot(feat, w1h_ref, w1l_ref))
        avu = _wdot(xr, vuh_ref, vul_ref) + vub_ref[...]
        h = jnp.tanh(avu[:, :D_ATT]) * jax.nn.sigmoid(avu[:, D_ATT:])
        l = jax.lax.dot_general(aw_ref[...], h, (((1,), (1,)), ((), ())),
                                preferred_element_type=F32,
                                precision=HIGHEST) + ab_ref[...]
        logits_ref[0, 0:1, pl.ds(t * TN + i * TS, TS)] = l

        m_t = jnp.max(l, axis=1, keepdims=True)
        m_old = m_ref[...]
        m_new = jnp.maximum(m_old, m_t)
        corr = jnp.exp(m_old - m_new)
        p = jnp.exp(l - m_new)
        s_ref[...] = s_ref[...] * corr + jnp.sum(p, axis=1, keepdims=True)
        pooled = jax.lax.dot_general(p, feat, (((1,), (0,)), ((), ())),
                                     preferred_element_type=F32)
        acc_ref[...] = acc_ref[...] * corr + pooled
        m_ref[...] = m_new

    @pl.when(t == NT - 1)
    def _():
        afeat_ref[0] = acc_ref[...] / s_ref[...]


def _stage2_body(logits_ref, afeat_ref, w2_ref, a2vw_ref, a2uw_ref, a2vub_ref,
                 a2w_ref, a2b_ref, a3vw_ref, a3uw_ref, a3vub_ref,
                 a3w_ref, a3b_ref,
                 clsw0_ref, clsw1_ref, clsb_ref, slw_ref, slb_ref,
                 a1_ref, a2_ref, out_ref, slide_ref,
                 w2h_ref, w2l_ref, vu2h_ref, vu2l_ref, vu3h_ref, vu3l_ref):
    _wsplit(w2h_ref, w2l_ref, w2_ref[...], 0)
    _wsplit(vu2h_ref, vu2l_ref, a2vw_ref[...], 0)
    _wsplit(vu2h_ref, vu2l_ref, a2uw_ref[...], D_ATT)
    _wsplit(vu3h_ref, vu3l_ref, a3vw_ref[...], 0)
    _wsplit(vu3h_ref, vu3l_ref, a3uw_ref[...], D_ATT)

    l = logits_ref[...]
    m = jnp.max(l, axis=1, keepdims=True)
    p = jnp.exp(l - m)
    a1_ref[...] = p / jnp.sum(p, axis=1, keepdims=True)

    afeat = afeat_ref[...]
    y = jax.nn.relu(_wdot(afeat, w2h_ref, w2l_ref))
    avu = _wdot(y, vu2h_ref, vu2l_ref) + a2vub_ref[...]
    h = jnp.tanh(avu[:, :D_ATT]) * jax.nn.sigmoid(avu[:, D_ATT:])
    a2p = jax.lax.dot_general(a2w_ref[...], h, (((1,), (1,)), ((), ())),
                              preferred_element_type=F32,
                              precision=HIGHEST) + a2b_ref[...]
    m2 = jnp.max(a2p, axis=1, keepdims=True)
    e2 = jnp.exp(a2p - m2)
    a2 = e2 / jnp.sum(e2, axis=1, keepdims=True)
    a2_ref[...] = a2

    afeat2 = jax.lax.dot_general(a2, afeat, (((1,), (0,)), ((), ())),
                                 preferred_element_type=F32,
                                 precision=HIGHEST)
    o0 = jnp.sum(afeat2 * clsw0_ref[...], axis=1, keepdims=True)
    o1 = jnp.sum(afeat2 * clsw1_ref[...], axis=1, keepdims=True)
    out_ref[...] = jnp.concatenate([o0, o1], axis=1) + clsb_ref[...]

    ii = jax.lax.broadcasted_iota(jnp.int32, (2, N2 + N3), 0)
    jj = jax.lax.broadcasted_iota(jnp.int32, (2, N2 + N3), 1)
    mix = jnp.where(((ii == 0) & (jj >= N2)) | ((ii == 1) & (jj < N2)),
                    0.5, 0.0)
    bag_a = jax.lax.dot_general(mix, a2, (((1,), (0,)), ((), ())),
                                preferred_element_type=F32,
                                precision=HIGHEST)
    bag_feat = jax.lax.dot_general(bag_a, afeat, (((1,), (0,)), ((), ())),
                                   preferred_element_type=F32,
                                   precision=HIGHEST)
    avu3 = _wdot(bag_feat, vu3h_ref, vu3l_ref) + a3vub_ref[...]
    h3 = jnp.tanh(avu3[:, :D_ATT]) * jax.nn.sigmoid(avu3[:, D_ATT:])
    a3p = jax.lax.dot_general(a3w_ref[...], h3, (((1,), (1,)), ((), ())),
                              preferred_element_type=F32,
                              precision=HIGHEST) + a3b_ref[...]
    m3 = jnp.max(a3p, axis=1, keepdims=True)
    e3 = jnp.exp(a3p - m3)
    a3 = e3 / jnp.sum(e3, axis=1, keepdims=True)
    fb = jax.lax.dot_general(a3, bag_feat, (((1,), (0,)), ((), ())),
                             preferred_element_type=F32,
                             precision=HIGHEST)
    slide_ref[...] = jax.lax.dot_general(
        fb, slw_ref[...], (((1,), (1,)), ((), ())),
        preferred_element_type=F32, precision=HIGHEST) + slb_ref[...]


def kernel(x, W_dr1, W_dr2, a1_Vw, a1_Vb, a1_Uw, a1_Ub, a1_w, a1_b,
           a2_Vw, a2_Vb, a2_Uw, a2_Ub, a2_w, a2_b,
           a3_Vw, a3_Vb, a3_Uw, a3_Ub, a3_w, a3_b,
           cls_w, cls_b, slide_w, slide_b):
    whole = lambda shape: pl.BlockSpec(shape, lambda b, t: (0,) * len(shape))

    stage1 = pl.pallas_call(
        _stage1_body,
        grid=(B, NT),
        in_specs=[
            pl.BlockSpec((1, 1, TN, D_FEAT), lambda b, t: (0, b, t, 0)),
            whole((D_INNER, D_FEAT)),
            whole((D_ATT, D_INNER)),
            whole((D_ATT, D_INNER)),
            whole((1, 2 * D_ATT)),
            whole((N1, D_ATT)),
            whole((1, N1)),
        ],
        out_specs=[
            pl.BlockSpec((1, 1, N), lambda b, t: (b, 0, 0)),
            pl.BlockSpec((1, 1, D_FEAT), lambda b, t: (b, 0, 0)),
        ],
        out_shape=[
            jax.ShapeDtypeStruct((B, 1, N), F32),
            jax.ShapeDtypeStruct((B, 1, D_FEAT), F32),
        ],
        scratch_shapes=[
            pltpu.VMEM((1, 1), F32),
            pltpu.VMEM((1, 1), F32),
            pltpu.VMEM((1, D_FEAT), F32),
            pltpu.VMEM((D_INNER, D_FEAT), F32),
            pltpu.VMEM((D_INNER, D_FEAT), BF),
            pltpu.VMEM((2 * D_ATT, D_INNER), F32),
            pltpu.VMEM((2 * D_ATT, D_INNER), BF),
        ],
        compiler_params=pltpu.CompilerParams(
            dimension_semantics=("arbitrary", "arbitrary")),
    )
    vub = jnp.concatenate([a1_Vb, a1_Ub], axis=0).reshape(1, 2 * D_ATT)
    logits, afeat = stage1(x, W_dr1, a1_Vw, a1_Uw, vub,
                           a1_w, a1_b.reshape(1, N1))

    logits = logits.reshape(B, N)
    afeat = afeat.reshape(B, D_FEAT)

    T = N2 + N3
    vub2 = jnp.concatenate([a2_Vb, a2_Ub], axis=0).reshape(1, 2 * D_ATT)
    vub3 = jnp.concatenate([a3_Vb, a3_Ub], axis=0).reshape(1, 2 * D_ATT)
    a1_out, a2_out, outputs, slide = pl.pallas_call(
        _stage2_body,
        out_shape=[
            jax.ShapeDtypeStruct((B, N), F32),
            jax.ShapeDtypeStruct((T, B), F32),
            jax.ShapeDtypeStruct((T, N_CLASS), F32),
            jax.ShapeDtypeStruct((1, N_CLASS), F32),
        ],
        scratch_shapes=[
            pltpu.VMEM((D_INNER, D_FEAT), F32),
            pltpu.VMEM((D_INNER, D_FEAT), BF),
            pltpu.VMEM((2 * D_ATT, D_INNER), F32),
            pltpu.VMEM((2 * D_ATT, D_INNER), BF),
            pltpu.VMEM((2 * D_ATT, D_FEAT), F32),
            pltpu.VMEM((2 * D_ATT, D_FEAT), BF),
        ],
    )(logits, afeat, W_dr2, a2_Vw, a2_Uw, vub2, a2_w, a2_b.reshape(T, 1),
      a3_Vw, a3_Uw, vub3, a3_w, a3_b.reshape(1, 1),
      cls_w[:, 0, :], cls_w[:, 1, :], cls_b, slide_w, slide_b.reshape(1, N_CLASS))

    A_1 = a1_out.reshape(B, N1, N)
    return (outputs, slide, A_1, a2_out[:N2], a2_out[N2:], a2_out)
